# Initial kernel scaffold; baseline (speedup 1.0000x reference)
import functools

import jax
import jax.numpy as jnp
from jax import lax
from jax.experimental import pallas as pl
from jax.experimental.pallas import tpu as pltpu
from jax.experimental.pallas import tpu_sc as plsc

N_NODES = 10000
D_IN = 256
D_HID = 128
D_OUT = 256

NC = 2
NS = 16
CHUNK = 128

N_PAD = 10240
E_PAD = 163840
DUMMY = N_NODES

ROW_BLK = 512
DEG_W = 16


def _vector_mesh():
    return plsc.VectorSubcoreMesh(core_axis_name="c", subcore_axis_name="s")


def _sc_degree(dst_pad, ones_hbm, zeros_hbm):
    @functools.partial(
        pl.kernel,
        mesh=_vector_mesh(),
        out_type=jax.ShapeDtypeStruct((NC, N_PAD, DEG_W), jnp.float32),
        scratch_types=[
            pltpu.VMEM((CHUNK,), jnp.int32),
            pltpu.VMEM((CHUNK, DEG_W), jnp.float32),
            pltpu.VMEM_SHARED((N_PAD, DEG_W), jnp.float32),
        ],
    )
    def k(dst_h, ones_h, zeros_h, out_h, dst_v, ones_v, acc_sh):
        cid = lax.axis_index("c")
        sid = lax.axis_index("s")
        rpt = N_PAD // NS
        pltpu.sync_copy(zeros_h.at[pl.ds(sid * rpt, rpt)],
                        acc_sh.at[pl.ds(sid * rpt, rpt)])
        pltpu.sync_copy(ones_h, ones_v)
        plsc.subcore_barrier()
        ept = E_PAD // (NC * NS)
        base = cid * (E_PAD // NC) + sid * ept

        @pl.loop(0, ept, step=CHUNK)
        def _(off):
            pltpu.sync_copy(dst_h.at[pl.ds(base + off, CHUNK)], dst_v)
            pltpu.sync_copy(ones_v, acc_sh.at[dst_v], add=True)

        plsc.subcore_barrier()
        pltpu.sync_copy(acc_sh.at[pl.ds(sid * rpt, rpt)],
                        out_h.at[cid, pl.ds(sid * rpt, rpt)])

    return k(dst_pad, ones_hbm, zeros_hbm)


def _sc_edge_pass(table, src_pad, dst_pad, zeros_hbm):
    @functools.partial(
        pl.kernel,
        mesh=_vector_mesh(),
        out_type=jax.ShapeDtypeStruct((NC, N_PAD, D_HID), jnp.float32),
        scratch_types=[
            pltpu.VMEM((CHUNK,), jnp.int32),
            pltpu.VMEM((CHUNK,), jnp.int32),
            pltpu.VMEM((CHUNK, D_HID), jnp.float32),
            pltpu.VMEM_SHARED((N_PAD, D_HID), jnp.float32),
            pltpu.SemaphoreType.DMA,
        ],
    )
    def k(tab_h, src_h, dst_h, zeros_h, out_h, src_v, dst_v, rows_v, acc_sh,
          sem):
        cid = lax.axis_index("c")
        sid = lax.axis_index("s")
        rpt = N_PAD // NS
        pltpu.sync_copy(zeros_h.at[pl.ds(sid * rpt, rpt)],
                        acc_sh.at[pl.ds(sid * rpt, rpt)])
        plsc.subcore_barrier()
        ept = E_PAD // (NC * NS)
        base = cid * (E_PAD // NC) + sid * ept

        @pl.loop(0, ept, step=CHUNK)
        def _(off):
            pltpu.sync_copy(src_h.at[pl.ds(base + off, CHUNK)], src_v)
            pltpu.sync_copy(dst_h.at[pl.ds(base + off, CHUNK)], dst_v)
            pltpu.async_copy(tab_h.at[src_v], rows_v, sem).wait()
            pltpu.sync_copy(rows_v, acc_sh.at[dst_v], add=True)

        plsc.subcore_barrier()
        pltpu.sync_copy(acc_sh.at[pl.ds(sid * rpt, rpt)],
                        out_h.at[cid, pl.ds(sid * rpt, rpt)])

    return k(table, src_pad, dst_pad, zeros_hbm)


def _tc_pre(x_pad, deg_parts, W0):
    def body(x_ref, dg_ref, w_ref, h_ref, dis_ref):
        cnt = dg_ref[0, :, 0:1] + dg_ref[1, :, 0:1]
        dis = lax.rsqrt(cnt + 1.0)
        h = jnp.dot(x_ref[...], w_ref[...], preferred_element_type=jnp.float32)
        h_ref[...] = h * dis
        dis_ref[...] = dis

    return pl.pallas_call(
        body,
        grid=(N_PAD // ROW_BLK,),
        in_specs=[
            pl.BlockSpec((ROW_BLK, D_IN), lambda i: (i, 0)),
            pl.BlockSpec((NC, ROW_BLK, DEG_W), lambda i: (0, i, 0)),
            pl.BlockSpec((D_IN, D_HID), lambda i: (0, 0)),
        ],
        out_specs=[
            pl.BlockSpec((ROW_BLK, D_HID), lambda i: (i, 0)),
            pl.BlockSpec((ROW_BLK, 1), lambda i: (i, 0)),
        ],
        out_shape=[
            jax.ShapeDtypeStruct((N_PAD, D_HID), jnp.float32),
            jax.ShapeDtypeStruct((N_PAD, 1), jnp.float32),
        ],
    )(x_pad, deg_parts, W0)


def _tc_mid(acc1, h0p, dis, b0, W1):
    def body(acc_ref, h0_ref, dis_ref, b0_ref, w_ref, out_ref):
        agg = acc_ref[0] + acc_ref[1] + h0_ref[...]
        dis = dis_ref[...]
        h1 = jnp.maximum(dis * agg + b0_ref[...], 0.0)
        w = w_ref[...]
        out_ref[0] = jnp.dot(h1, w[:, :D_HID],
                             preferred_element_type=jnp.float32) * dis
        out_ref[1] = jnp.dot(h1, w[:, D_HID:],
                             preferred_element_type=jnp.float32) * dis

    return pl.pallas_call(
        body,
        grid=(N_PAD // ROW_BLK,),
        in_specs=[
            pl.BlockSpec((NC, ROW_BLK, D_HID), lambda i: (0, i, 0)),
            pl.BlockSpec((ROW_BLK, D_HID), lambda i: (i, 0)),
            pl.BlockSpec((ROW_BLK, 1), lambda i: (i, 0)),
            pl.BlockSpec((1, D_HID), lambda i: (0, 0)),
            pl.BlockSpec((D_HID, D_OUT), lambda i: (0, 0)),
        ],
        out_specs=pl.BlockSpec((2, ROW_BLK, D_HID), lambda i: (0, i, 0)),
        out_shape=jax.ShapeDtypeStruct((2, N_PAD, D_HID), jnp.float32),
    )(acc1, h0p, dis, b0, W1)


def _tc_post(accA, accB, h1p, dis, b1):
    def body(a_ref, b_ref, h_ref, dis_ref, b1_ref, out_ref):
        dis = dis_ref[...]
        out_ref[:, :D_HID] = dis * (a_ref[0] + a_ref[1] + h_ref[0]) \
            + b1_ref[:, :D_HID]
        out_ref[:, D_HID:] = dis * (b_ref[0] + b_ref[1] + h_ref[1]) \
            + b1_ref[:, D_HID:]

    return pl.pallas_call(
        body,
        grid=(N_PAD // ROW_BLK,),
        in_specs=[
            pl.BlockSpec((NC, ROW_BLK, D_HID), lambda i: (0, i, 0)),
            pl.BlockSpec((NC, ROW_BLK, D_HID), lambda i: (0, i, 0)),
            pl.BlockSpec((2, ROW_BLK, D_HID), lambda i: (0, i, 0)),
            pl.BlockSpec((ROW_BLK, 1), lambda i: (i, 0)),
            pl.BlockSpec((1, D_OUT), lambda i: (0, 0)),
        ],
        out_specs=pl.BlockSpec((ROW_BLK, D_OUT), lambda i: (i, 0)),
        out_shape=jax.ShapeDtypeStruct((N_PAD, D_OUT), jnp.float32),
    )(accA, accB, h1p, dis, b1)


def kernel(x, edge_index, W0, b0, W1, b1):
    src = edge_index[0]
    dst = edge_index[1]
    pad_e = E_PAD - src.shape[0]
    src_pad = jnp.concatenate(
        [src, jnp.full((pad_e,), DUMMY, dtype=jnp.int32)])
    dst_pad = jnp.concatenate(
        [dst, jnp.full((pad_e,), DUMMY, dtype=jnp.int32)])
    x_pad = jnp.pad(x, ((0, N_PAD - N_NODES), (0, 0)))

    ones_hbm = jnp.ones((CHUNK, DEG_W), dtype=jnp.float32)
    zeros_deg = jnp.zeros((N_PAD, DEG_W), dtype=jnp.float32)
    zeros_feat = jnp.zeros((N_PAD, D_HID), dtype=jnp.float32)

    deg_parts = _sc_degree(dst_pad, ones_hbm, zeros_deg)
    h0p, dis = _tc_pre(x_pad, deg_parts, W0)
    acc1 = _sc_edge_pass(h0p, src_pad, dst_pad, zeros_feat)
    h1p = _tc_mid(acc1, h0p, dis, b0.reshape(1, D_HID), W1)
    accA = _sc_edge_pass(h1p[0], src_pad, dst_pad, zeros_feat)
    accB = _sc_edge_pass(h1p[1], src_pad, dst_pad, zeros_feat)
    out = _tc_post(accA, accB, h1p, dis, b1.reshape(1, D_OUT))
    return out[:N_NODES]


# speedup vs baseline: 4.7510x; 4.7510x over previous
import functools

import jax
import jax.numpy as jnp
from jax import lax
from jax.experimental import pallas as pl
from jax.experimental.pallas import tpu as pltpu
from jax.experimental.pallas import tpu_sc as plsc

N_NODES = 10000
D_IN = 256
D_HID = 128
D_OUT = 256

NC = 2
NS = 16
CHUNK = 128

N_PAD = 10240
E_PAD = 163840
DUMMY = N_NODES

ROW_BLK = 512
DEG_W = 128


def _vector_mesh():
    return plsc.VectorSubcoreMesh(core_axis_name="c", subcore_axis_name="s")


def _sc_degree(dst_pad, ones_hbm, zeros_hbm):
    @functools.partial(
        pl.kernel,
        mesh=_vector_mesh(),
        out_type=jax.ShapeDtypeStruct((NC, N_PAD, DEG_W), jnp.float32),
        scratch_types=[
            pltpu.VMEM((CHUNK,), jnp.int32),
            pltpu.VMEM((CHUNK, DEG_W), jnp.float32),
            pltpu.VMEM_SHARED((N_PAD, DEG_W), jnp.float32),
        ],
    )
    def k(dst_h, ones_h, zeros_h, out_h, dst_v, ones_v, acc_sh):
        cid = lax.axis_index("c")
        sid = lax.axis_index("s")
        rpt = N_PAD // NS
        pltpu.sync_copy(zeros_h.at[pl.ds(sid * rpt, rpt)],
                        acc_sh.at[pl.ds(sid * rpt, rpt)])
        pltpu.sync_copy(ones_h, ones_v)
        plsc.subcore_barrier()
        ept = E_PAD // (NC * NS)
        base = cid * (E_PAD // NC) + sid * ept

        @pl.loop(0, ept, step=CHUNK)
        def _(off):
            pltpu.sync_copy(dst_h.at[pl.ds(base + off, CHUNK)], dst_v)
            pltpu.sync_copy(ones_v, acc_sh.at[dst_v], add=True)

        plsc.subcore_barrier()
        pltpu.sync_copy(acc_sh.at[pl.ds(sid * rpt, rpt)],
                        out_h.at[cid, pl.ds(sid * rpt, rpt)])

    return k(dst_pad, ones_hbm, zeros_hbm)


def _sc_edge_pass(table, src_pad, dst_pad, zeros_hbm):
    @functools.partial(
        pl.kernel,
        mesh=_vector_mesh(),
        out_type=jax.ShapeDtypeStruct((NC, N_PAD, D_HID), jnp.float32),
        scratch_types=[
            pltpu.VMEM((CHUNK,), jnp.int32),
            pltpu.VMEM((CHUNK,), jnp.int32),
            pltpu.VMEM((CHUNK, D_HID), jnp.float32),
            pltpu.VMEM_SHARED((N_PAD, D_HID), jnp.float32),
            pltpu.SemaphoreType.DMA,
        ],
    )
    def k(tab_h, src_h, dst_h, zeros_h, out_h, src_v, dst_v, rows_v, acc_sh,
          sem):
        cid = lax.axis_index("c")
        sid = lax.axis_index("s")
        rpt = N_PAD // NS
        pltpu.sync_copy(zeros_h.at[pl.ds(sid * rpt, rpt)],
                        acc_sh.at[pl.ds(sid * rpt, rpt)])
        plsc.subcore_barrier()
        ept = E_PAD // (NC * NS)
        base = cid * (E_PAD // NC) + sid * ept

        @pl.loop(0, ept, step=CHUNK)
        def _(off):
            pltpu.sync_copy(src_h.at[pl.ds(base + off, CHUNK)], src_v)
            pltpu.sync_copy(dst_h.at[pl.ds(base + off, CHUNK)], dst_v)
            pltpu.async_copy(tab_h.at[src_v], rows_v, sem).wait()
            pltpu.sync_copy(rows_v, acc_sh.at[dst_v], add=True)

        plsc.subcore_barrier()
        pltpu.sync_copy(acc_sh.at[pl.ds(sid * rpt, rpt)],
                        out_h.at[cid, pl.ds(sid * rpt, rpt)])

    return k(table, src_pad, dst_pad, zeros_hbm)


def _tc_pre(x_pad, deg_parts, W0):
    def body(x_ref, dg_ref, w_ref, h_ref, dis_ref):
        cnt = dg_ref[0, :, 0:1] + dg_ref[1, :, 0:1]
        dis = lax.rsqrt(cnt + 1.0)
        h = jnp.dot(x_ref[...], w_ref[...], preferred_element_type=jnp.float32)
        h_ref[...] = h * dis
        dis_ref[...] = dis

    return pl.pallas_call(
        body,
        grid=(N_PAD // ROW_BLK,),
        in_specs=[
            pl.BlockSpec((ROW_BLK, D_IN), lambda i: (i, 0)),
            pl.BlockSpec((NC, ROW_BLK, DEG_W), lambda i: (0, i, 0)),
            pl.BlockSpec((D_IN, D_HID), lambda i: (0, 0)),
        ],
        out_specs=[
            pl.BlockSpec((ROW_BLK, D_HID), lambda i: (i, 0)),
            pl.BlockSpec((ROW_BLK, 1), lambda i: (i, 0)),
        ],
        out_shape=[
            jax.ShapeDtypeStruct((N_PAD, D_HID), jnp.float32),
            jax.ShapeDtypeStruct((N_PAD, 1), jnp.float32),
        ],
    )(x_pad, deg_parts, W0)


def _tc_mid(acc1, h0p, dis, b0, W1):
    def body(acc_ref, h0_ref, dis_ref, b0_ref, w_ref, out_ref):
        agg = acc_ref[0] + acc_ref[1] + h0_ref[...]
        dis = dis_ref[...]
        h1 = jnp.maximum(dis * agg + b0_ref[...], 0.0)
        w = w_ref[...]
        out_ref[0] = jnp.dot(h1, w[:, :D_HID],
                             preferred_element_type=jnp.float32) * dis
        out_ref[1] = jnp.dot(h1, w[:, D_HID:],
                             preferred_element_type=jnp.float32) * dis

    return pl.pallas_call(
        body,
        grid=(N_PAD // ROW_BLK,),
        in_specs=[
            pl.BlockSpec((NC, ROW_BLK, D_HID), lambda i: (0, i, 0)),
            pl.BlockSpec((ROW_BLK, D_HID), lambda i: (i, 0)),
            pl.BlockSpec((ROW_BLK, 1), lambda i: (i, 0)),
            pl.BlockSpec((1, D_HID), lambda i: (0, 0)),
            pl.BlockSpec((D_HID, D_OUT), lambda i: (0, 0)),
        ],
        out_specs=pl.BlockSpec((2, ROW_BLK, D_HID), lambda i: (0, i, 0)),
        out_shape=jax.ShapeDtypeStruct((2, N_PAD, D_HID), jnp.float32),
    )(acc1, h0p, dis, b0, W1)


def _tc_post(accA, accB, h1p, dis, b1):
    def body(a_ref, b_ref, h_ref, dis_ref, b1_ref, out_ref):
        dis = dis_ref[...]
        out_ref[:, :D_HID] = dis * (a_ref[0] + a_ref[1] + h_ref[0]) \
            + b1_ref[:, :D_HID]
        out_ref[:, D_HID:] = dis * (b_ref[0] + b_ref[1] + h_ref[1]) \
            + b1_ref[:, D_HID:]

    return pl.pallas_call(
        body,
        grid=(N_PAD // ROW_BLK,),
        in_specs=[
            pl.BlockSpec((NC, ROW_BLK, D_HID), lambda i: (0, i, 0)),
            pl.BlockSpec((NC, ROW_BLK, D_HID), lambda i: (0, i, 0)),
            pl.BlockSpec((2, ROW_BLK, D_HID), lambda i: (0, i, 0)),
            pl.BlockSpec((ROW_BLK, 1), lambda i: (i, 0)),
            pl.BlockSpec((1, D_OUT), lambda i: (0, 0)),
        ],
        out_specs=pl.BlockSpec((ROW_BLK, D_OUT), lambda i: (i, 0)),
        out_shape=jax.ShapeDtypeStruct((N_PAD, D_OUT), jnp.float32),
    )(accA, accB, h1p, dis, b1)


def kernel(x, edge_index, W0, b0, W1, b1):
    src = edge_index[0]
    dst = edge_index[1]
    pad_e = E_PAD - src.shape[0]
    src_pad = jnp.concatenate(
        [src, jnp.full((pad_e,), DUMMY, dtype=jnp.int32)])
    dst_pad = jnp.concatenate(
        [dst, jnp.full((pad_e,), DUMMY, dtype=jnp.int32)])
    x_pad = jnp.pad(x, ((0, N_PAD - N_NODES), (0, 0)))

    ones_hbm = jnp.ones((CHUNK, DEG_W), dtype=jnp.float32)
    zeros_feat = jnp.zeros((N_PAD, D_HID), dtype=jnp.float32)

    deg_parts = _sc_degree(dst_pad, ones_hbm, zeros_feat)
    h0p, dis = _tc_pre(x_pad, deg_parts, W0)
    acc1 = _sc_edge_pass(h0p, src_pad, dst_pad, zeros_feat)
    h1p = _tc_mid(acc1, h0p, dis, b0.reshape(1, D_HID), W1)
    accA = _sc_edge_pass(h1p[0], src_pad, dst_pad, zeros_feat)
    accB = _sc_edge_pass(h1p[1], src_pad, dst_pad, zeros_feat)
    out = _tc_post(accA, accB, h1p, dis, b1.reshape(1, D_OUT))
    return out[:N_NODES]


# speedup vs baseline: 5.6494x; 1.1891x over previous
import functools

import jax
import jax.numpy as jnp
from jax import lax
from jax.experimental import pallas as pl
from jax.experimental.pallas import tpu as pltpu
from jax.experimental.pallas import tpu_sc as plsc

N_NODES = 10000
D_IN = 256
D_HID = 128
D_OUT = 256

NC = 2
NS = 16
CHUNK = 128

N_PAD = 10240
E_PAD = 163840
DUMMY = N_NODES

ROW_BLK = 512
DEG_W = 128


def _vector_mesh():
    return plsc.VectorSubcoreMesh(core_axis_name="c", subcore_axis_name="s")


def _sc_degree(dst_pad, ones_hbm, zeros_hbm):
    @functools.partial(
        pl.kernel,
        mesh=_vector_mesh(),
        out_type=jax.ShapeDtypeStruct((NC, N_PAD, DEG_W), jnp.float32),
        scratch_types=[
            pltpu.VMEM((CHUNK,), jnp.int32),
            pltpu.VMEM((CHUNK, DEG_W), jnp.float32),
            pltpu.VMEM_SHARED((N_PAD, DEG_W), jnp.float32),
        ],
    )
    def k(dst_h, ones_h, zeros_h, out_h, dst_v, ones_v, acc_sh):
        cid = lax.axis_index("c")
        sid = lax.axis_index("s")
        rpt = N_PAD // NS
        pltpu.sync_copy(zeros_h.at[pl.ds(sid * rpt, rpt)],
                        acc_sh.at[pl.ds(sid * rpt, rpt)])
        pltpu.sync_copy(ones_h, ones_v)
        plsc.subcore_barrier()
        ept = E_PAD // (NC * NS)
        base = cid * (E_PAD // NC) + sid * ept

        @pl.loop(0, ept, step=CHUNK)
        def _(off):
            pltpu.sync_copy(dst_h.at[pl.ds(base + off, CHUNK)], dst_v)
            pltpu.sync_copy(ones_v, acc_sh.at[dst_v], add=True)

        plsc.subcore_barrier()
        pltpu.sync_copy(acc_sh.at[pl.ds(sid * rpt, rpt)],
                        out_h.at[cid, pl.ds(sid * rpt, rpt)])

    return k(dst_pad, ones_hbm, zeros_hbm)


N_CHUNKS = E_PAD // (NC * NS) // CHUNK


def _sc_edge_pass(table, src3, dst3, zeros_hbm):
    @functools.partial(
        pl.kernel,
        mesh=_vector_mesh(),
        out_type=jax.ShapeDtypeStruct((NC, N_PAD, D_HID), jnp.float32),
        scratch_types=[
            pltpu.VMEM((N_CHUNKS, CHUNK), jnp.int32),
            pltpu.VMEM((N_CHUNKS, CHUNK), jnp.int32),
            pltpu.VMEM((CHUNK, D_HID), jnp.float32),
            pltpu.VMEM((CHUNK, D_HID), jnp.float32),
            pltpu.VMEM_SHARED((N_PAD, D_HID), jnp.float32),
            pltpu.SemaphoreType.DMA,
            pltpu.SemaphoreType.DMA,
        ],
    )
    def k(tab_h, src_h, dst_h, zeros_h, out_h, src_v, dst_v, rows0, rows1,
          acc_sh, sem0, sem1):
        cid = lax.axis_index("c")
        sid = lax.axis_index("s")
        wid = cid * NS + sid
        rpt = N_PAD // NS
        pltpu.sync_copy(zeros_h.at[pl.ds(sid * rpt, rpt)],
                        acc_sh.at[pl.ds(sid * rpt, rpt)])
        pltpu.sync_copy(src_h.at[wid], src_v)
        pltpu.sync_copy(dst_h.at[wid], dst_v)
        plsc.subcore_barrier()

        g0 = pltpu.async_copy(tab_h.at[src_v.at[0]], rows0, sem0)

        @pl.loop(0, N_CHUNKS - 2, step=2)
        def _(j):
            pltpu.async_copy(tab_h.at[src_v.at[j + 1]], rows1, sem1)
            pltpu.make_async_copy(tab_h.at[src_v.at[j]], rows0, sem0).wait()
            pltpu.sync_copy(rows0, acc_sh.at[dst_v.at[j]], add=True)
            pltpu.async_copy(tab_h.at[src_v.at[j + 2]], rows0, sem0)
            pltpu.make_async_copy(tab_h.at[src_v.at[j + 1]], rows1,
                                  sem1).wait()
            pltpu.sync_copy(rows1, acc_sh.at[dst_v.at[j + 1]], add=True)

        pltpu.async_copy(tab_h.at[src_v.at[N_CHUNKS - 1]], rows1, sem1)
        pltpu.make_async_copy(tab_h.at[src_v.at[N_CHUNKS - 2]], rows0,
                              sem0).wait()
        pltpu.sync_copy(rows0, acc_sh.at[dst_v.at[N_CHUNKS - 2]], add=True)
        pltpu.make_async_copy(tab_h.at[src_v.at[N_CHUNKS - 1]], rows1,
                              sem1).wait()
        pltpu.sync_copy(rows1, acc_sh.at[dst_v.at[N_CHUNKS - 1]], add=True)

        plsc.subcore_barrier()
        pltpu.sync_copy(acc_sh.at[pl.ds(sid * rpt, rpt)],
                        out_h.at[cid, pl.ds(sid * rpt, rpt)])

    return k(table, src3, dst3, zeros_hbm)


def _tc_pre(x_pad, deg_parts, W0):
    def body(x_ref, dg_ref, w_ref, h_ref, dis_ref):
        cnt = dg_ref[0, :, 0:1] + dg_ref[1, :, 0:1]
        dis = lax.rsqrt(cnt + 1.0)
        h = jnp.dot(x_ref[...], w_ref[...], preferred_element_type=jnp.float32)
        h_ref[...] = h * dis
        dis_ref[...] = dis

    return pl.pallas_call(
        body,
        grid=(N_PAD // ROW_BLK,),
        in_specs=[
            pl.BlockSpec((ROW_BLK, D_IN), lambda i: (i, 0)),
            pl.BlockSpec((NC, ROW_BLK, DEG_W), lambda i: (0, i, 0)),
            pl.BlockSpec((D_IN, D_HID), lambda i: (0, 0)),
        ],
        out_specs=[
            pl.BlockSpec((ROW_BLK, D_HID), lambda i: (i, 0)),
            pl.BlockSpec((ROW_BLK, 1), lambda i: (i, 0)),
        ],
        out_shape=[
            jax.ShapeDtypeStruct((N_PAD, D_HID), jnp.float32),
            jax.ShapeDtypeStruct((N_PAD, 1), jnp.float32),
        ],
    )(x_pad, deg_parts, W0)


def _tc_mid(acc1, h0p, dis, b0, W1):
    def body(acc_ref, h0_ref, dis_ref, b0_ref, w_ref, out_ref):
        agg = acc_ref[0] + acc_ref[1] + h0_ref[...]
        dis = dis_ref[...]
        h1 = jnp.maximum(dis * agg + b0_ref[...], 0.0)
        w = w_ref[...]
        out_ref[0] = jnp.dot(h1, w[:, :D_HID],
                             preferred_element_type=jnp.float32) * dis
        out_ref[1] = jnp.dot(h1, w[:, D_HID:],
                             preferred_element_type=jnp.float32) * dis

    return pl.pallas_call(
        body,
        grid=(N_PAD // ROW_BLK,),
        in_specs=[
            pl.BlockSpec((NC, ROW_BLK, D_HID), lambda i: (0, i, 0)),
            pl.BlockSpec((ROW_BLK, D_HID), lambda i: (i, 0)),
            pl.BlockSpec((ROW_BLK, 1), lambda i: (i, 0)),
            pl.BlockSpec((1, D_HID), lambda i: (0, 0)),
            pl.BlockSpec((D_HID, D_OUT), lambda i: (0, 0)),
        ],
        out_specs=pl.BlockSpec((2, ROW_BLK, D_HID), lambda i: (0, i, 0)),
        out_shape=jax.ShapeDtypeStruct((2, N_PAD, D_HID), jnp.float32),
    )(acc1, h0p, dis, b0, W1)


def _tc_post(accA, accB, h1p, dis, b1):
    def body(a_ref, b_ref, h_ref, dis_ref, b1_ref, out_ref):
        dis = dis_ref[...]
        out_ref[:, :D_HID] = dis * (a_ref[0] + a_ref[1] + h_ref[0]) \
            + b1_ref[:, :D_HID]
        out_ref[:, D_HID:] = dis * (b_ref[0] + b_ref[1] + h_ref[1]) \
            + b1_ref[:, D_HID:]

    return pl.pallas_call(
        body,
        grid=(N_PAD // ROW_BLK,),
        in_specs=[
            pl.BlockSpec((NC, ROW_BLK, D_HID), lambda i: (0, i, 0)),
            pl.BlockSpec((NC, ROW_BLK, D_HID), lambda i: (0, i, 0)),
            pl.BlockSpec((2, ROW_BLK, D_HID), lambda i: (0, i, 0)),
            pl.BlockSpec((ROW_BLK, 1), lambda i: (i, 0)),
            pl.BlockSpec((1, D_OUT), lambda i: (0, 0)),
        ],
        out_specs=pl.BlockSpec((ROW_BLK, D_OUT), lambda i: (i, 0)),
        out_shape=jax.ShapeDtypeStruct((N_PAD, D_OUT), jnp.float32),
    )(accA, accB, h1p, dis, b1)


def kernel(x, edge_index, W0, b0, W1, b1):
    src = edge_index[0]
    dst = edge_index[1]
    pad_e = E_PAD - src.shape[0]
    src_pad = jnp.concatenate(
        [src, jnp.full((pad_e,), DUMMY, dtype=jnp.int32)])
    dst_pad = jnp.concatenate(
        [dst, jnp.full((pad_e,), DUMMY, dtype=jnp.int32)])
    x_pad = jnp.pad(x, ((0, N_PAD - N_NODES), (0, 0)))
    src3 = src_pad.reshape(NC * NS, N_CHUNKS, CHUNK)
    dst3 = dst_pad.reshape(NC * NS, N_CHUNKS, CHUNK)

    ones_hbm = jnp.ones((CHUNK, DEG_W), dtype=jnp.float32)
    zeros_feat = jnp.zeros((N_PAD, D_HID), dtype=jnp.float32)

    deg_parts = _sc_degree(dst_pad, ones_hbm, zeros_feat)
    h0p, dis = _tc_pre(x_pad, deg_parts, W0)
    acc1 = _sc_edge_pass(h0p, src3, dst3, zeros_feat)
    h1p = _tc_mid(acc1, h0p, dis, b0.reshape(1, D_HID), W1)
    accA = _sc_edge_pass(h1p[0], src3, dst3, zeros_feat)
    accB = _sc_edge_pass(h1p[1], src3, dst3, zeros_feat)
    out = _tc_post(accA, accB, h1p, dis, b1.reshape(1, D_OUT))
    return out[:N_NODES]


# speedup vs baseline: 14.9208x; 2.6411x over previous
import functools

import jax
import jax.numpy as jnp
from jax import lax
from jax.experimental import pallas as pl
from jax.experimental.pallas import tpu as pltpu
from jax.experimental.pallas import tpu_sc as plsc

N_NODES = 10000
D_IN = 256
D_HID = 128
D_OUT = 256

NC = 2
NS = 16
CHUNK = 128

N_PAD = 10240
E_PAD = 163840
DUMMY = N_NODES

ROW_BLK = 512
DEG_W = 128


def _vector_mesh():
    return plsc.VectorSubcoreMesh(core_axis_name="c", subcore_axis_name="s")


def _sc_degree(dst_pad, ones_hbm, zeros_hbm):
    @functools.partial(
        pl.kernel,
        mesh=_vector_mesh(),
        out_type=jax.ShapeDtypeStruct((NC, N_PAD, DEG_W), jnp.float32),
        scratch_types=[
            pltpu.VMEM((CHUNK,), jnp.int32),
            pltpu.VMEM((CHUNK, DEG_W), jnp.float32),
            pltpu.VMEM_SHARED((N_PAD, DEG_W), jnp.float32),
        ],
    )
    def k(dst_h, ones_h, zeros_h, out_h, dst_v, ones_v, acc_sh):
        cid = lax.axis_index("c")
        sid = lax.axis_index("s")
        rpt = N_PAD // NS
        pltpu.sync_copy(zeros_h.at[pl.ds(sid * rpt, rpt)],
                        acc_sh.at[pl.ds(sid * rpt, rpt)])
        pltpu.sync_copy(ones_h, ones_v)
        plsc.subcore_barrier()
        ept = E_PAD // (NC * NS)
        base = cid * (E_PAD // NC) + sid * ept

        @pl.loop(0, ept, step=CHUNK)
        def _(off):
            pltpu.sync_copy(dst_h.at[pl.ds(base + off, CHUNK)], dst_v)
            pltpu.sync_copy(ones_v, acc_sh.at[dst_v], add=True)

        plsc.subcore_barrier()
        pltpu.sync_copy(acc_sh.at[pl.ds(sid * rpt, rpt)],
                        out_h.at[cid, pl.ds(sid * rpt, rpt)])

    return k(dst_pad, ones_hbm, zeros_hbm)


N_CHUNKS = E_PAD // (NC * NS) // CHUNK


def _sc_edge_pass(table, src3, dst3, zeros_hbm):
    @functools.partial(
        pl.kernel,
        mesh=_vector_mesh(),
        out_type=jax.ShapeDtypeStruct((NC, N_PAD, D_HID), jnp.float32),
        scratch_types=[
            pltpu.VMEM((N_CHUNKS, CHUNK), jnp.int32),
            pltpu.VMEM((N_CHUNKS, CHUNK), jnp.int32),
            pltpu.VMEM((CHUNK, D_HID), jnp.float32),
            pltpu.VMEM((CHUNK, D_HID), jnp.float32),
            pltpu.VMEM_SHARED((N_PAD, D_HID), jnp.float32),
            pltpu.SemaphoreType.DMA,
            pltpu.SemaphoreType.DMA,
        ],
    )
    def k(tab_h, src_h, dst_h, zeros_h, out_h, src_v, dst_v, rows0, rows1,
          acc_sh, sem0, sem1):
        cid = lax.axis_index("c")
        sid = lax.axis_index("s")
        wid = cid * NS + sid
        rpt = N_PAD // NS
        pltpu.sync_copy(zeros_h.at[pl.ds(sid * rpt, rpt)],
                        acc_sh.at[pl.ds(sid * rpt, rpt)])
        pltpu.sync_copy(src_h.at[wid], src_v)
        pltpu.sync_copy(dst_h.at[wid], dst_v)
        plsc.subcore_barrier()

        g0 = pltpu.async_copy(tab_h.at[src_v.at[0]], rows0, sem0)

        @pl.loop(0, N_CHUNKS - 2, step=2)
        def _(j):
            pltpu.async_copy(tab_h.at[src_v.at[j + 1]], rows1, sem1)
            pltpu.make_async_copy(tab_h.at[src_v.at[j]], rows0, sem0).wait()
            pltpu.sync_copy(rows0, acc_sh.at[dst_v.at[j]], add=True)
            pltpu.async_copy(tab_h.at[src_v.at[j + 2]], rows0, sem0)
            pltpu.make_async_copy(tab_h.at[src_v.at[j + 1]], rows1,
                                  sem1).wait()
            pltpu.sync_copy(rows1, acc_sh.at[dst_v.at[j + 1]], add=True)

        pltpu.async_copy(tab_h.at[src_v.at[N_CHUNKS - 1]], rows1, sem1)
        pltpu.make_async_copy(tab_h.at[src_v.at[N_CHUNKS - 2]], rows0,
                              sem0).wait()
        pltpu.sync_copy(rows0, acc_sh.at[dst_v.at[N_CHUNKS - 2]], add=True)
        pltpu.make_async_copy(tab_h.at[src_v.at[N_CHUNKS - 1]], rows1,
                              sem1).wait()
        pltpu.sync_copy(rows1, acc_sh.at[dst_v.at[N_CHUNKS - 1]], add=True)

        plsc.subcore_barrier()
        pltpu.sync_copy(acc_sh.at[pl.ds(sid * rpt, rpt)],
                        out_h.at[cid, pl.ds(sid * rpt, rpt)])

    return k(table, src3, dst3, zeros_hbm)


def _tc_pre(x_pad, deg_parts, W0):
    def body(x_ref, dg_ref, w_ref, h_ref, dis_ref):
        cnt = dg_ref[0, :, 0:1] + dg_ref[1, :, 0:1]
        dis = lax.rsqrt(cnt + 1.0)
        h = jnp.dot(x_ref[...], w_ref[...], preferred_element_type=jnp.float32)
        h_ref[...] = h * dis
        dis_ref[...] = dis

    return pl.pallas_call(
        body,
        grid=(N_PAD // ROW_BLK,),
        in_specs=[
            pl.BlockSpec((ROW_BLK, D_IN), lambda i: (i, 0)),
            pl.BlockSpec((NC, ROW_BLK, DEG_W), lambda i: (0, i, 0)),
            pl.BlockSpec((D_IN, D_HID), lambda i: (0, 0)),
        ],
        out_specs=[
            pl.BlockSpec((ROW_BLK, D_HID), lambda i: (i, 0)),
            pl.BlockSpec((ROW_BLK, 1), lambda i: (i, 0)),
        ],
        out_shape=[
            jax.ShapeDtypeStruct((N_PAD, D_HID), jnp.float32),
            jax.ShapeDtypeStruct((N_PAD, 1), jnp.float32),
        ],
    )(x_pad, deg_parts, W0)


def _tc_mid(acc1, h0p, dis, b0, W1):
    def body(acc_ref, h0_ref, dis_ref, b0_ref, w_ref, out_ref):
        agg = acc_ref[0] + acc_ref[1] + h0_ref[...]
        dis = dis_ref[...]
        h1 = jnp.maximum(dis * agg + b0_ref[...], 0.0)
        w = w_ref[...]
        out_ref[0] = jnp.dot(h1, w[:, :D_HID],
                             preferred_element_type=jnp.float32) * dis
        out_ref[1] = jnp.dot(h1, w[:, D_HID:],
                             preferred_element_type=jnp.float32) * dis

    return pl.pallas_call(
        body,
        grid=(N_PAD // ROW_BLK,),
        in_specs=[
            pl.BlockSpec((NC, ROW_BLK, D_HID), lambda i: (0, i, 0)),
            pl.BlockSpec((ROW_BLK, D_HID), lambda i: (i, 0)),
            pl.BlockSpec((ROW_BLK, 1), lambda i: (i, 0)),
            pl.BlockSpec((1, D_HID), lambda i: (0, 0)),
            pl.BlockSpec((D_HID, D_OUT), lambda i: (0, 0)),
        ],
        out_specs=pl.BlockSpec((2, ROW_BLK, D_HID), lambda i: (0, i, 0)),
        out_shape=jax.ShapeDtypeStruct((2, N_PAD, D_HID), jnp.float32),
    )(acc1, h0p, dis, b0, W1)


def _tc_post(accA, accB, h1p, dis, b1):
    def body(a_ref, b_ref, h_ref, dis_ref, b1_ref, out_ref):
        dis = dis_ref[...]
        out_ref[:, :D_HID] = dis * (a_ref[0] + a_ref[1] + h_ref[0]) \
            + b1_ref[:, :D_HID]
        out_ref[:, D_HID:] = dis * (b_ref[0] + b_ref[1] + h_ref[1]) \
            + b1_ref[:, D_HID:]

    return pl.pallas_call(
        body,
        grid=(N_PAD // ROW_BLK,),
        in_specs=[
            pl.BlockSpec((NC, ROW_BLK, D_HID), lambda i: (0, i, 0)),
            pl.BlockSpec((NC, ROW_BLK, D_HID), lambda i: (0, i, 0)),
            pl.BlockSpec((2, ROW_BLK, D_HID), lambda i: (0, i, 0)),
            pl.BlockSpec((ROW_BLK, 1), lambda i: (i, 0)),
            pl.BlockSpec((1, D_OUT), lambda i: (0, 0)),
        ],
        out_specs=pl.BlockSpec((ROW_BLK, D_OUT), lambda i: (i, 0)),
        out_shape=jax.ShapeDtypeStruct((N_PAD, D_OUT), jnp.float32),
    )(accA, accB, h1p, dis, b1)


def kernel(x, edge_index, W0, b0, W1, b1):
    src = edge_index[0]
    dst = edge_index[1]
    pad_e = E_PAD - src.shape[0]
    pad_idx = DUMMY + jnp.arange(pad_e, dtype=jnp.int32) % (N_PAD - N_NODES)
    src_pad = jnp.concatenate([src, pad_idx])
    dst_pad = jnp.concatenate([dst, pad_idx])
    x_pad = jnp.pad(x, ((0, N_PAD - N_NODES), (0, 0)))
    src3 = src_pad.reshape(NC * NS, N_CHUNKS, CHUNK)
    dst3 = dst_pad.reshape(NC * NS, N_CHUNKS, CHUNK)

    ones_hbm = jnp.ones((CHUNK, DEG_W), dtype=jnp.float32)
    zeros_feat = jnp.zeros((N_PAD, D_HID), dtype=jnp.float32)

    deg_parts = _sc_degree(dst_pad, ones_hbm, zeros_feat)
    h0p, dis = _tc_pre(x_pad, deg_parts, W0)
    acc1 = _sc_edge_pass(h0p, src3, dst3, zeros_feat)
    h1p = _tc_mid(acc1, h0p, dis, b0.reshape(1, D_HID), W1)
    accA = _sc_edge_pass(h1p[0], src3, dst3, zeros_feat)
    accB = _sc_edge_pass(h1p[1], src3, dst3, zeros_feat)
    out = _tc_post(accA, accB, h1p, dis, b1.reshape(1, D_OUT))
    return out[:N_NODES]


# speedup vs baseline: 19.0329x; 1.2756x over previous
import functools

import jax
import jax.numpy as jnp
from jax import lax
from jax.experimental import pallas as pl
from jax.experimental.pallas import tpu as pltpu
from jax.experimental.pallas import tpu_sc as plsc

N_NODES = 10000
D_IN = 256
D_HID = 128
D_OUT = 256

NC = 2
NS = 16
CHUNK = 128

N_PAD = 10240
E_PAD = 163840
DUMMY = N_NODES

ROW_BLK = 512
DEG_W = 128


def _vector_mesh():
    return plsc.VectorSubcoreMesh(core_axis_name="c", subcore_axis_name="s")


def _sc_degree(dst_pad, ones_hbm, zeros_hbm):
    @functools.partial(
        pl.kernel,
        mesh=_vector_mesh(),
        out_type=jax.ShapeDtypeStruct((NC, N_PAD, DEG_W), jnp.float32),
        scratch_types=[
            pltpu.VMEM((CHUNK,), jnp.int32),
            pltpu.VMEM((CHUNK, DEG_W), jnp.float32),
            pltpu.VMEM_SHARED((N_PAD, DEG_W), jnp.float32),
        ],
    )
    def k(dst_h, ones_h, zeros_h, out_h, dst_v, ones_v, acc_sh):
        cid = lax.axis_index("c")
        sid = lax.axis_index("s")
        rpt = N_PAD // NS
        pltpu.sync_copy(zeros_h.at[pl.ds(sid * rpt, rpt)],
                        acc_sh.at[pl.ds(sid * rpt, rpt)])
        pltpu.sync_copy(ones_h, ones_v)
        plsc.subcore_barrier()
        ept = E_PAD // (NC * NS)
        base = cid * (E_PAD // NC) + sid * ept

        @pl.loop(0, ept, step=CHUNK)
        def _(off):
            pltpu.sync_copy(dst_h.at[pl.ds(base + off, CHUNK)], dst_v)
            pltpu.sync_copy(ones_v, acc_sh.at[dst_v], add=True)

        plsc.subcore_barrier()
        pltpu.sync_copy(acc_sh.at[pl.ds(sid * rpt, rpt)],
                        out_h.at[cid, pl.ds(sid * rpt, rpt)])

    return k(dst_pad, ones_hbm, zeros_hbm)


N_CHUNKS = E_PAD // (NC * NS) // CHUNK


def _sc_edge_pass(table, src3, dst3, zeros_hbm):
    @functools.partial(
        pl.kernel,
        mesh=_vector_mesh(),
        out_type=jax.ShapeDtypeStruct((NC, N_PAD, D_HID), jnp.float32),
        scratch_types=[
            pltpu.VMEM((N_CHUNKS, CHUNK), jnp.int32),
            pltpu.VMEM((N_CHUNKS, CHUNK), jnp.int32),
            pltpu.VMEM((CHUNK, D_HID), jnp.float32),
            pltpu.VMEM((CHUNK, D_HID), jnp.float32),
            pltpu.VMEM_SHARED((N_PAD, D_HID), jnp.float32),
            pltpu.SemaphoreType.DMA,
            pltpu.SemaphoreType.DMA,
        ],
    )
    def k(tab_h, src_h, dst_h, zeros_h, out_h, src_v, dst_v, rows0, rows1,
          acc_sh, sem0, sem1):
        cid = lax.axis_index("c")
        sid = lax.axis_index("s")
        wid = cid * NS + sid
        rpt = N_PAD // NS
        pltpu.sync_copy(zeros_h.at[pl.ds(sid * rpt, rpt)],
                        acc_sh.at[pl.ds(sid * rpt, rpt)])
        pltpu.sync_copy(src_h.at[wid], src_v)
        pltpu.sync_copy(dst_h.at[wid], dst_v)
        plsc.subcore_barrier()

        g0 = pltpu.async_copy(tab_h.at[src_v.at[0]], rows0, sem0)

        @pl.loop(0, N_CHUNKS - 2, step=2)
        def _(j):
            pltpu.async_copy(tab_h.at[src_v.at[j + 1]], rows1, sem1)
            pltpu.make_async_copy(tab_h.at[src_v.at[j]], rows0, sem0).wait()
            pltpu.sync_copy(rows0, acc_sh.at[dst_v.at[j]], add=True)
            pltpu.async_copy(tab_h.at[src_v.at[j + 2]], rows0, sem0)
            pltpu.make_async_copy(tab_h.at[src_v.at[j + 1]], rows1,
                                  sem1).wait()
            pltpu.sync_copy(rows1, acc_sh.at[dst_v.at[j + 1]], add=True)

        pltpu.async_copy(tab_h.at[src_v.at[N_CHUNKS - 1]], rows1, sem1)
        pltpu.make_async_copy(tab_h.at[src_v.at[N_CHUNKS - 2]], rows0,
                              sem0).wait()
        pltpu.sync_copy(rows0, acc_sh.at[dst_v.at[N_CHUNKS - 2]], add=True)
        pltpu.make_async_copy(tab_h.at[src_v.at[N_CHUNKS - 1]], rows1,
                              sem1).wait()
        pltpu.sync_copy(rows1, acc_sh.at[dst_v.at[N_CHUNKS - 1]], add=True)

        plsc.subcore_barrier()
        pltpu.sync_copy(acc_sh.at[pl.ds(sid * rpt, rpt)],
                        out_h.at[cid, pl.ds(sid * rpt, rpt)])

    return k(table, src3, dst3, zeros_hbm)


def _tc_mm0(x_pad, W0):
    def body(x_ref, w_ref, h_ref):
        h_ref[...] = jnp.dot(x_ref[...], w_ref[...],
                             preferred_element_type=jnp.float32)

    return pl.pallas_call(
        body,
        grid=(N_PAD // ROW_BLK,),
        in_specs=[
            pl.BlockSpec((ROW_BLK, D_IN), lambda i: (i, 0)),
            pl.BlockSpec((D_IN, D_HID), lambda i: (0, 0)),
        ],
        out_specs=pl.BlockSpec((ROW_BLK, D_HID), lambda i: (i, 0)),
        out_shape=jax.ShapeDtypeStruct((N_PAD, D_HID), jnp.float32),
    )(x_pad, W0)


def _tc_scale(h0, deg_parts):
    def body(h0_ref, dg_ref, h_ref, dis_ref):
        cnt = dg_ref[0, :, 0:1] + dg_ref[1, :, 0:1]
        dis = lax.rsqrt(cnt + 1.0)
        h_ref[...] = h0_ref[...] * dis
        dis_ref[...] = dis

    return pl.pallas_call(
        body,
        grid=(N_PAD // ROW_BLK,),
        in_specs=[
            pl.BlockSpec((ROW_BLK, D_HID), lambda i: (i, 0)),
            pl.BlockSpec((NC, ROW_BLK, DEG_W), lambda i: (0, i, 0)),
        ],
        out_specs=[
            pl.BlockSpec((ROW_BLK, D_HID), lambda i: (i, 0)),
            pl.BlockSpec((ROW_BLK, 1), lambda i: (i, 0)),
        ],
        out_shape=[
            jax.ShapeDtypeStruct((N_PAD, D_HID), jnp.float32),
            jax.ShapeDtypeStruct((N_PAD, 1), jnp.float32),
        ],
    )(h0, deg_parts)


def _tc_mid(acc1, h0p, dis, b0):
    def body(acc_ref, h0_ref, dis_ref, b0_ref, out_ref):
        agg = acc_ref[0] + acc_ref[1] + h0_ref[...]
        dis = dis_ref[...]
        h1 = jnp.maximum(dis * agg + b0_ref[...], 0.0)
        out_ref[...] = h1 * dis

    return pl.pallas_call(
        body,
        grid=(N_PAD // ROW_BLK,),
        in_specs=[
            pl.BlockSpec((NC, ROW_BLK, D_HID), lambda i: (0, i, 0)),
            pl.BlockSpec((ROW_BLK, D_HID), lambda i: (i, 0)),
            pl.BlockSpec((ROW_BLK, 1), lambda i: (i, 0)),
            pl.BlockSpec((1, D_HID), lambda i: (0, 0)),
        ],
        out_specs=pl.BlockSpec((ROW_BLK, D_HID), lambda i: (i, 0)),
        out_shape=jax.ShapeDtypeStruct((N_PAD, D_HID), jnp.float32),
    )(acc1, h0p, dis, b0)


def _tc_post(acc2, g1, dis, W1, b1):
    def body(acc_ref, g_ref, dis_ref, w_ref, b1_ref, out_ref):
        agg2 = dis_ref[...] * (acc_ref[0] + acc_ref[1] + g_ref[...])
        out_ref[...] = jnp.dot(agg2, w_ref[...],
                               preferred_element_type=jnp.float32) \
            + b1_ref[...]

    return pl.pallas_call(
        body,
        grid=(N_PAD // ROW_BLK,),
        in_specs=[
            pl.BlockSpec((NC, ROW_BLK, D_HID), lambda i: (0, i, 0)),
            pl.BlockSpec((ROW_BLK, D_HID), lambda i: (i, 0)),
            pl.BlockSpec((ROW_BLK, 1), lambda i: (i, 0)),
            pl.BlockSpec((D_HID, D_OUT), lambda i: (0, 0)),
            pl.BlockSpec((1, D_OUT), lambda i: (0, 0)),
        ],
        out_specs=pl.BlockSpec((ROW_BLK, D_OUT), lambda i: (i, 0)),
        out_shape=jax.ShapeDtypeStruct((N_PAD, D_OUT), jnp.float32),
    )(acc2, g1, dis, W1, b1)


def kernel(x, edge_index, W0, b0, W1, b1):
    src = edge_index[0]
    dst = edge_index[1]
    pad_e = E_PAD - src.shape[0]
    pad_idx = DUMMY + jnp.arange(pad_e, dtype=jnp.int32) % (N_PAD - N_NODES)
    src_pad = jnp.concatenate([src, pad_idx])
    dst_pad = jnp.concatenate([dst, pad_idx])
    x_pad = jnp.pad(x, ((0, N_PAD - N_NODES), (0, 0)))
    src3 = src_pad.reshape(NC * NS, N_CHUNKS, CHUNK)
    dst3 = dst_pad.reshape(NC * NS, N_CHUNKS, CHUNK)

    ones_hbm = jnp.ones((CHUNK, DEG_W), dtype=jnp.float32)
    zeros_feat = jnp.zeros((N_PAD, D_HID), dtype=jnp.float32)

    deg_parts = _sc_degree(dst_pad, ones_hbm, zeros_feat)
    h0 = _tc_mm0(x_pad, W0)
    h0p, dis = _tc_scale(h0, deg_parts)
    acc1 = _sc_edge_pass(h0p, src3, dst3, zeros_feat)
    g1 = _tc_mid(acc1, h0p, dis, b0.reshape(1, D_HID))
    acc2 = _sc_edge_pass(g1, src3, dst3, zeros_feat)
    out = _tc_post(acc2, g1, dis, W1, b1.reshape(1, D_OUT))
    return out[:N_NODES]


# speedup vs baseline: 22.8233x; 1.1992x over previous
import functools

import jax
import jax.numpy as jnp
from jax import lax
from jax.experimental import pallas as pl
from jax.experimental.pallas import tpu as pltpu
from jax.experimental.pallas import tpu_sc as plsc

N_NODES = 10000
D_IN = 256
D_HID = 128
D_OUT = 256

NC = 2
NS = 16
CHUNK = 128

N_PAD = 10240
E_PAD = 163840
DUMMY = N_NODES

ROW_BLK = 1024
OUT_BLK = 1000
DEG_W = 128
N_CHUNKS = E_PAD // (NC * NS) // CHUNK


def _vector_mesh():
    return plsc.VectorSubcoreMesh(core_axis_name="c", subcore_axis_name="s")


def _sc_degree(ei3, ones_hbm, zeros_hbm):
    @functools.partial(
        pl.kernel,
        mesh=_vector_mesh(),
        out_type=jax.ShapeDtypeStruct((NC, N_PAD, DEG_W), jnp.float32),
        scratch_types=[
            pltpu.VMEM((N_CHUNKS, CHUNK), jnp.int32),
            pltpu.VMEM((CHUNK, DEG_W), jnp.float32),
            pltpu.VMEM_SHARED((N_PAD, DEG_W), jnp.float32),
            pltpu.SemaphoreType.DMA,
        ],
    )
    def k(ei_h, ones_h, zeros_h, out_h, dst_v, ones_v, acc_sh, sem):
        cid = lax.axis_index("c")
        sid = lax.axis_index("s")
        wid = cid * NS + sid
        rpt = N_PAD // NS
        pltpu.sync_copy(zeros_h.at[pl.ds(sid * rpt, rpt)],
                        acc_sh.at[pl.ds(sid * rpt, rpt)])
        pltpu.sync_copy(ones_h, ones_v)
        pltpu.sync_copy(ei_h.at[1, wid], dst_v)
        plsc.subcore_barrier()

        @pl.loop(0, N_CHUNKS)
        def _(j):
            pltpu.async_copy(ones_v, acc_sh.at[dst_v.at[j]], sem, add=True)

        @pl.loop(0, N_CHUNKS)
        def _(j):
            pltpu.make_async_copy(ones_v, acc_sh.at[dst_v.at[j]], sem).wait()

        plsc.subcore_barrier()
        pltpu.sync_copy(acc_sh.at[pl.ds(sid * rpt, rpt)],
                        out_h.at[cid, pl.ds(sid * rpt, rpt)])

    return k(ei3, ones_hbm, zeros_hbm)


def _sc_edge_pass(table, ei3, zeros_hbm):
    @functools.partial(
        pl.kernel,
        mesh=_vector_mesh(),
        out_type=jax.ShapeDtypeStruct((NC, N_PAD, D_HID), jnp.float32),
        scratch_types=[
            pltpu.VMEM((N_CHUNKS, CHUNK), jnp.int32),
            pltpu.VMEM((N_CHUNKS, CHUNK), jnp.int32),
            pltpu.VMEM((CHUNK, D_HID), jnp.float32),
            pltpu.VMEM((CHUNK, D_HID), jnp.float32),
            pltpu.VMEM_SHARED((N_PAD, D_HID), jnp.float32),
            pltpu.SemaphoreType.DMA,
            pltpu.SemaphoreType.DMA,
        ],
    )
    def k(tab_h, ei_h, zeros_h, out_h, src_v, dst_v, rows0, rows1,
          acc_sh, sem0, sem1):
        cid = lax.axis_index("c")
        sid = lax.axis_index("s")
        wid = cid * NS + sid
        rpt = N_PAD // NS
        pltpu.sync_copy(zeros_h.at[pl.ds(sid * rpt, rpt)],
                        acc_sh.at[pl.ds(sid * rpt, rpt)])
        pltpu.sync_copy(ei_h.at[0, wid], src_v)
        pltpu.sync_copy(ei_h.at[1, wid], dst_v)
        plsc.subcore_barrier()

        g0 = pltpu.async_copy(tab_h.at[src_v.at[0]], rows0, sem0)

        @pl.loop(0, N_CHUNKS - 2, step=2)
        def _(j):
            pltpu.async_copy(tab_h.at[src_v.at[j + 1]], rows1, sem1)
            pltpu.make_async_copy(tab_h.at[src_v.at[j]], rows0, sem0).wait()
            pltpu.sync_copy(rows0, acc_sh.at[dst_v.at[j]], add=True)
            pltpu.async_copy(tab_h.at[src_v.at[j + 2]], rows0, sem0)
            pltpu.make_async_copy(tab_h.at[src_v.at[j + 1]], rows1,
                                  sem1).wait()
            pltpu.sync_copy(rows1, acc_sh.at[dst_v.at[j + 1]], add=True)

        pltpu.async_copy(tab_h.at[src_v.at[N_CHUNKS - 1]], rows1, sem1)
        pltpu.make_async_copy(tab_h.at[src_v.at[N_CHUNKS - 2]], rows0,
                              sem0).wait()
        pltpu.sync_copy(rows0, acc_sh.at[dst_v.at[N_CHUNKS - 2]], add=True)
        pltpu.make_async_copy(tab_h.at[src_v.at[N_CHUNKS - 1]], rows1,
                              sem1).wait()
        pltpu.sync_copy(rows1, acc_sh.at[dst_v.at[N_CHUNKS - 1]], add=True)

        plsc.subcore_barrier()
        pltpu.sync_copy(acc_sh.at[pl.ds(sid * rpt, rpt)],
                        out_h.at[cid, pl.ds(sid * rpt, rpt)])

    return k(table, ei3, zeros_hbm)


def _tc_mm0(x_pad, W0):
    def body(x_ref, w_ref, h_ref):
        h_ref[...] = jnp.dot(x_ref[...], w_ref[...],
                             preferred_element_type=jnp.float32)

    return pl.pallas_call(
        body,
        grid=(N_PAD // ROW_BLK,),
        in_specs=[
            pl.BlockSpec((ROW_BLK, D_IN), lambda i: (i, 0)),
            pl.BlockSpec((D_IN, D_HID), lambda i: (0, 0)),
        ],
        out_specs=pl.BlockSpec((ROW_BLK, D_HID), lambda i: (i, 0)),
        out_shape=jax.ShapeDtypeStruct((N_PAD, D_HID), jnp.float32),
    )(x_pad, W0)


def _tc_scale(h0, deg_parts):
    def body(h0_ref, dg_ref, h_ref, dis_ref):
        cnt = dg_ref[0, :, 0:1] + dg_ref[1, :, 0:1]
        dis = lax.rsqrt(cnt + 1.0)
        h_ref[...] = h0_ref[...] * dis
        dis_ref[...] = dis

    return pl.pallas_call(
        body,
        grid=(N_PAD // ROW_BLK,),
        in_specs=[
            pl.BlockSpec((ROW_BLK, D_HID), lambda i: (i, 0)),
            pl.BlockSpec((NC, ROW_BLK, DEG_W), lambda i: (0, i, 0)),
        ],
        out_specs=[
            pl.BlockSpec((ROW_BLK, D_HID), lambda i: (i, 0)),
            pl.BlockSpec((ROW_BLK, 1), lambda i: (i, 0)),
        ],
        out_shape=[
            jax.ShapeDtypeStruct((N_PAD, D_HID), jnp.float32),
            jax.ShapeDtypeStruct((N_PAD, 1), jnp.float32),
        ],
    )(h0, deg_parts)


def _tc_mid(acc1, h0p, dis, b0):
    def body(acc_ref, h0_ref, dis_ref, b0_ref, out_ref):
        agg = acc_ref[0] + acc_ref[1] + h0_ref[...]
        dis = dis_ref[...]
        h1 = jnp.maximum(dis * agg + b0_ref[...], 0.0)
        out_ref[...] = h1 * dis

    return pl.pallas_call(
        body,
        grid=(N_PAD // ROW_BLK,),
        in_specs=[
            pl.BlockSpec((NC, ROW_BLK, D_HID), lambda i: (0, i, 0)),
            pl.BlockSpec((ROW_BLK, D_HID), lambda i: (i, 0)),
            pl.BlockSpec((ROW_BLK, 1), lambda i: (i, 0)),
            pl.BlockSpec((1, D_HID), lambda i: (0, 0)),
        ],
        out_specs=pl.BlockSpec((ROW_BLK, D_HID), lambda i: (i, 0)),
        out_shape=jax.ShapeDtypeStruct((N_PAD, D_HID), jnp.float32),
    )(acc1, h0p, dis, b0)


def _tc_post(acc2, g1, dis, W1, b1):
    def body(acc_ref, g_ref, dis_ref, w_ref, b1_ref, out_ref):
        agg2 = dis_ref[...] * (acc_ref[0] + acc_ref[1] + g_ref[...])
        out_ref[...] = jnp.dot(agg2, w_ref[...],
                               preferred_element_type=jnp.float32) \
            + b1_ref[...]

    return pl.pallas_call(
        body,
        grid=(N_NODES // OUT_BLK,),
        in_specs=[
            pl.BlockSpec((NC, OUT_BLK, D_HID), lambda i: (0, i, 0)),
            pl.BlockSpec((OUT_BLK, D_HID), lambda i: (i, 0)),
            pl.BlockSpec((OUT_BLK, 1), lambda i: (i, 0)),
            pl.BlockSpec((D_HID, D_OUT), lambda i: (0, 0)),
            pl.BlockSpec((1, D_OUT), lambda i: (0, 0)),
        ],
        out_specs=pl.BlockSpec((OUT_BLK, D_OUT), lambda i: (i, 0)),
        out_shape=jax.ShapeDtypeStruct((N_NODES, D_OUT), jnp.float32),
    )(acc2, g1, dis, W1, b1)


def kernel(x, edge_index, W0, b0, W1, b1):
    pad_e = E_PAD - edge_index.shape[1]
    pad_idx = DUMMY + jnp.arange(pad_e, dtype=jnp.int32) % (N_PAD - N_NODES)
    pad2 = jnp.broadcast_to(pad_idx, (2, pad_e))
    ei3 = jnp.concatenate([edge_index, pad2], axis=1).reshape(
        2, NC * NS, N_CHUNKS, CHUNK)
    x_pad = jnp.pad(x, ((0, N_PAD - N_NODES), (0, 0)))

    ones_hbm = jnp.ones((CHUNK, DEG_W), dtype=jnp.float32)
    zeros_feat = jnp.zeros((N_PAD, D_HID), dtype=jnp.float32)

    deg_parts = _sc_degree(ei3, ones_hbm, zeros_feat)
    h0 = _tc_mm0(x_pad, W0)
    h0p, dis = _tc_scale(h0, deg_parts)
    acc1 = _sc_edge_pass(h0p, ei3, zeros_feat)
    g1 = _tc_mid(acc1, h0p, dis, b0.reshape(1, D_HID))
    acc2 = _sc_edge_pass(g1, ei3, zeros_feat)
    return _tc_post(acc2, g1, dis, W1, b1.reshape(1, D_OUT))
